# Initial kernel scaffold; baseline (speedup 1.0000x reference)
import jax
import jax.numpy as jnp
from jax.experimental import pallas as pl


def kernel(x, in_proj_w, in_proj_b, out_proj_w, out_proj_b, g1_w, g1_b, g2_w, g2_b):
    raise NotImplementedError("write your pallas kernel here")


# speedup vs baseline: 5.1016x; 5.1016x over previous
import functools

import jax
import jax.numpy as jnp
from jax.experimental import pallas as pl
from jax.experimental.pallas import tpu as pltpu

D = 1024
NH = 16
HD = 64
BATCH = 4
S = 2048
K_SEL = max(1, int(S * 0.1))
K_PAD = 256
QB = 256
NQ = S // QB
NEG = -1e30


def _cumsum_lanes(a, n):
    sh = 1
    while sh < n:
        shifted = jnp.concatenate(
            [jnp.zeros((1, sh), jnp.int32), a[:, : n - sh]], axis=1)
        a = a + shifted
        sh *= 2
    return a


def _body(x_ref, xq_ref, wqkv_ref, bqkv_ref, wo_ref, bo_ref,
          g1_ref, g1b_ref, g2_ref, g2b_ref,
          out_ref, ksel_ref, vsel_ref):
    qi = pl.program_id(1)
    f32 = jnp.float32

    @pl.when(qi == 0)
    def _phase_a():
        xb = x_ref[0]
        hgt = jax.lax.dot_general(g1_ref[...], xb, (((1,), (1,)), ((), ())),
                                  preferred_element_type=f32)
        hgt = jnp.maximum(hgt + g1b_ref[...], 0.0)
        logit = jax.lax.dot_general(g2_ref[...], hgt, (((1,), (0,)), ((), ())),
                                    preferred_element_type=f32)
        logit = logit + g2b_ref[0, 0]
        imp = 1.0 / (1.0 + jnp.exp(-logit))
        bits = jax.lax.bitcast_convert_type(imp, jnp.int32)

        def bs(_, carry):
            lo, hi = carry
            mid = (lo + hi) // 2
            cnt = jnp.sum((bits >= mid).astype(jnp.int32))
            take = cnt >= K_SEL
            return (jnp.where(take, mid, lo), jnp.where(take, hi, mid))

        t, _ = jax.lax.fori_loop(
            0, 31, bs, (jnp.int32(0), jnp.int32(0x3F800001)))

        gt = bits > t
        tie = bits == t
        n_gt = jnp.sum(gt.astype(jnp.int32))
        r = K_SEL - n_gt
        tie_rank = _cumsum_lanes(tie.astype(jnp.int32), S)
        sel = gt | (tie & (tie_rank <= r))
        selr = _cumsum_lanes(sel.astype(jnp.int32), S) - 1
        selr = jnp.where(sel, selr, -1)

        rows = jax.lax.broadcasted_iota(jnp.int32, (K_PAD, S), 0)
        p = (rows == selr).astype(f32)
        x_sel = jax.lax.dot_general(p, xb, (((1,), (0,)), ((), ())),
                                    preferred_element_type=f32)
        wk = wqkv_ref[D:2 * D, :]
        wv = wqkv_ref[2 * D:3 * D, :]
        ksel_ref[...] = jax.lax.dot_general(
            x_sel, wk, (((1,), (1,)), ((), ())),
            preferred_element_type=f32) + bqkv_ref[1:2, :]
        vsel_ref[...] = jax.lax.dot_general(
            x_sel, wv, (((1,), (1,)), ((), ())),
            preferred_element_type=f32) + bqkv_ref[2:3, :]

    xq = xq_ref[0]
    wq = wqkv_ref[0:D, :]
    q = jax.lax.dot_general(xq, wq, (((1,), (1,)), ((), ())),
                            preferred_element_type=f32) + bqkv_ref[0:1, :]
    ksel = ksel_ref[...]
    vsel = vsel_ref[...]
    col = jax.lax.broadcasted_iota(jnp.int32, (QB, K_PAD), 1)
    pad_bias = jnp.where(col < K_SEL, 0.0, NEG)

    outs = []
    for h in range(NH):
        sl = slice(h * HD, (h + 1) * HD)
        s = jax.lax.dot_general(q[:, sl], ksel[:, sl], (((1,), (1,)), ((), ())),
                                preferred_element_type=f32)
        s = s * (1.0 / (HD ** 0.5)) + pad_bias
        m = jnp.max(s, axis=1, keepdims=True)
        e = jnp.exp(s - m)
        l = jnp.sum(e, axis=1, keepdims=True)
        oh = jax.lax.dot_general(e, vsel[:, sl], (((1,), (0,)), ((), ())),
                                 preferred_element_type=f32) / l
        outs.append(oh)
    o = jnp.concatenate(outs, axis=1)
    res = jax.lax.dot_general(o, wo_ref[...], (((1,), (1,)), ((), ())),
                              preferred_element_type=f32) + bo_ref[...]
    out_ref[0] = res


@jax.jit
def kernel(x, in_proj_w, in_proj_b, out_proj_w, out_proj_b,
           g1_w, g1_b, g2_w, g2_b):
    bqkv = in_proj_b.reshape(3, D)
    bo = out_proj_b.reshape(1, D)
    g1b = g1_b.reshape(D // 4, 1)
    g2b = g2_b.reshape(1, 1)

    grid = (BATCH, NQ)
    out = pl.pallas_call(
        _body,
        grid=grid,
        in_specs=[
            pl.BlockSpec((1, S, D), lambda b, q: (b, 0, 0)),
            pl.BlockSpec((1, QB, D), lambda b, q: (b, q, 0)),
            pl.BlockSpec((3 * D, D), lambda b, q: (0, 0)),
            pl.BlockSpec((3, D), lambda b, q: (0, 0)),
            pl.BlockSpec((D, D), lambda b, q: (0, 0)),
            pl.BlockSpec((1, D), lambda b, q: (0, 0)),
            pl.BlockSpec((D // 4, D), lambda b, q: (0, 0)),
            pl.BlockSpec((D // 4, 1), lambda b, q: (0, 0)),
            pl.BlockSpec((1, D // 4), lambda b, q: (0, 0)),
            pl.BlockSpec((1, 1), lambda b, q: (0, 0)),
        ],
        out_specs=pl.BlockSpec((1, QB, D), lambda b, q: (b, q, 0)),
        out_shape=jax.ShapeDtypeStruct((BATCH, S, D), jnp.float32),
        scratch_shapes=[
            pltpu.VMEM((K_PAD, D), jnp.float32),
            pltpu.VMEM((K_PAD, D), jnp.float32),
        ],
        compiler_params=pltpu.CompilerParams(
            dimension_semantics=("arbitrary", "arbitrary")),
    )(x, x, in_proj_w, bqkv, out_proj_w, bo, g1_w, g1b, g2_w, g2b)
    return out
